# baseline (speedup 1.0000x reference)
import jax
import jax.numpy as jnp
from jax.experimental import pallas as pl
from jax.experimental.pallas import tpu as pltpu

HID = 4096
NE = 64
CH = 1024
NBUF = 3


def _router_body(x_hbm, w_ref, b_ref, o_hbm, xbuf, obuf, insem, outsem):
    w = w_ref[...]
    bb = b_ref[...]
    nch = x_hbm.shape[0] // CH
    ngrp, ntail = nch // NBUF, nch % NBUF

    def read(i, s):
        return pltpu.make_async_copy(
            x_hbm.at[pl.ds(i * CH, CH)], xbuf.at[s], insem.at[s]
        )

    def write(i, s):
        return pltpu.make_async_copy(
            obuf.at[s], o_hbm.at[pl.ds(i * CH, CH)], outsem.at[s]
        )

    def compute(s):
        x = xbuf[s]
        logits = jax.lax.dot_general(
            x, w, (((1,), (1,)), ((), ())),
            preferred_element_type=jnp.float32,
        ) + bb
        m = jnp.max(logits, axis=-1, keepdims=True)
        e = jnp.exp(logits - m)
        return e / jnp.sum(e, axis=-1, keepdims=True)

    for s in range(NBUF):
        read(s, s).start()

    def group(g, _):
        for s in range(NBUF):
            i = g * NBUF + s
            read(i, s).wait()
            res = compute(s)

            @pl.when(i + NBUF < nch)
            def _():
                read(i + NBUF, s).start()

            @pl.when(g > 0)
            def _():
                write(i - NBUF, s).wait()

            obuf[s] = res
            write(i, s).start()
        return _

    jax.lax.fori_loop(0, ngrp, group, None)

    for t in range(ntail):
        i = ngrp * NBUF + t
        s = i % NBUF
        read(i, s).wait()
        res = compute(s)
        write(i - NBUF, s).wait()
        obuf[s] = res
        write(i, s).start()

    for i in range(nch - NBUF, nch):
        write(i, i % NBUF).wait()


def kernel(x, W, b):
    tokens = x.shape[0]
    return pl.pallas_call(
        _router_body,
        in_specs=[
            pl.BlockSpec(memory_space=pl.ANY),
            pl.BlockSpec((NE, HID), lambda: (0, 0)),
            pl.BlockSpec((1, NE), lambda: (0, 0)),
        ],
        out_specs=pl.BlockSpec(memory_space=pl.ANY),
        out_shape=jax.ShapeDtypeStruct((tokens, NE), jnp.float32),
        scratch_shapes=[
            pltpu.VMEM((NBUF, CH, HID), jnp.float32),
            pltpu.VMEM((NBUF, CH, NE), jnp.float32),
            pltpu.SemaphoreType.DMA((NBUF,)),
            pltpu.SemaphoreType.DMA((NBUF,)),
        ],
    )(x, W, b.reshape(1, NE))


# speedup vs baseline: 1.0237x; 1.0237x over previous
import jax
import jax.numpy as jnp
from jax.experimental import pallas as pl
from jax.experimental.pallas import tpu as pltpu

HID = 4096
NE = 64
BT = 1024


def _router_body(x_ref, w_ref, b_ref, o_ref):
    x = x_ref[...]
    w = w_ref[...]
    logits = jax.lax.dot_general(
        x, w, (((1,), (1,)), ((), ())), preferred_element_type=jnp.float32
    )
    logits = logits + b_ref[...]
    m = jnp.max(logits, axis=-1, keepdims=True)
    e = jnp.exp(logits - m)
    o_ref[...] = e / jnp.sum(e, axis=-1, keepdims=True)


def kernel(x, W, b):
    tokens = x.shape[0]
    return pl.pallas_call(
        _router_body,
        grid=(tokens // BT,),
        in_specs=[
            pl.BlockSpec((BT, HID), lambda i: (i, 0)),
            pl.BlockSpec((NE, HID), lambda i: (0, 0)),
            pl.BlockSpec((1, NE), lambda i: (0, 0)),
        ],
        out_specs=pl.BlockSpec((BT, NE), lambda i: (i, 0)),
        out_shape=jax.ShapeDtypeStruct((tokens, NE), jnp.float32),
        compiler_params=pltpu.CompilerParams(
            dimension_semantics=("parallel",),
        ),
    )(x, W, b.reshape(1, NE))
